# baseline (speedup 1.0000x reference)
import functools

import numpy as np

import jax
import jax.numpy as jnp
from jax import lax
from jax.experimental import pallas as pl
from jax.experimental.pallas import tpu as pltpu
from jax.experimental.pallas import tpu_sc as plsc

_NC = 2
_NS = 16
_NW = _NC * _NS

_C = 128
_S = 8
_L = 16


def _pack_table(table):
    d = table.shape[1]
    w = np.arange(d // 2)
    perm_lo = 2 * _L * (w // _L) + w % _L
    tb = jax.lax.bitcast_convert_type(table.astype(jnp.bfloat16), jnp.uint16)
    lo = tb[:, perm_lo].astype(jnp.uint32)
    hi = tb[:, perm_lo + _L].astype(jnp.uint32)
    return jax.lax.bitcast_convert_type((hi << 16) | lo, jnp.int32)


@functools.partial(jax.jit, static_argnums=(2, 3))
def _gather_flat(idx2d, packed, b, d):
    dw = d // 2
    b_per_w = b // _NW
    n_chunks = b_per_w // _C
    n_super = n_chunks // _S

    mesh = plsc.VectorSubcoreMesh(core_axis_name="c", subcore_axis_name="s")

    @functools.partial(
        pl.kernel,
        mesh=mesh,
        out_type=jax.ShapeDtypeStruct((b, d), jnp.int32),
        scratch_types=[
            pltpu.VMEM((2, _S, _C), jnp.int32),
            pltpu.VMEM((2, _C, dw), jnp.int32),
            pltpu.VMEM((2, _C, d), jnp.int32),
        ]
        + [pltpu.SemaphoreType.DMA] * 6,
        compiler_params=pltpu.CompilerParams(use_tc_tiling_on_sc=False),
    )
    def k(idx_hbm, tab_hbm, out_hbm, idx_v, rows_b, rows_f, *sems):
        gb = sems[0:2]
        of = sems[2:4]
        is_ = sems[4:6]

        wid = lax.axis_index("s") * _NC + lax.axis_index("c")
        crow0 = wid * n_chunks

        def fire_idx(g, p):
            pltpu.async_copy(idx_hbm.at[pl.ds(crow0 + g * _S, _S)],
                             idx_v.at[p], is_[p])

        def wait_idx(p):
            pltpu.make_async_copy(idx_hbm.at[pl.ds(0, _S)], idx_v.at[p],
                                  is_[p]).wait()

        def fire_gather(p, jj, a):
            pltpu.async_copy(tab_hbm.at[idx_v.at[p, jj]], rows_b.at[a],
                             gb[a])

        def wait_gather(a):
            pltpu.make_async_copy(tab_hbm.at[pl.ds(0, _C)], rows_b.at[a],
                                  gb[a]).wait()

        def fire_out(a, crow):
            off = pl.multiple_of(crow * _C, _C)
            pltpu.async_copy(rows_f.at[a], out_hbm.at[pl.ds(off, _C)], of[a])

        def wait_out(a):
            pltpu.make_async_copy(rows_f.at[a], out_hbm.at[pl.ds(0, _C)],
                                  of[a]).wait()

        def convert(a):
            def rows4(rr, carry):
                for u in range(4):
                    r = rr * 4 + u
                    for kk in range(dw // _L):
                        w = rows_b[a, r, pl.ds(kk * _L, _L)]
                        rows_f[a, r, pl.ds(2 * kk * _L, _L)] = w << 16
                        rows_f[a, r, pl.ds((2 * kk + 1) * _L, _L)] = (
                            w & jnp.int32(-65536))
                return carry

            lax.fori_loop(0, _C // 4, rows4, 0)

        def superblock(p, crow, g, first=False, last=False):
            if not last:
                fire_idx(jnp.minimum(g + 1, n_super - 1), p ^ 1)
            for j in range(_S):
                a = j % 2
                if not (first and j < 2):
                    wait_out(a)
                wait_gather(a)
                convert(a)
                fire_out(a, crow + j)
                if j == _S - 3 and not last:
                    wait_idx(p ^ 1)
                if j < _S - 2:
                    fire_gather(p, j + 2, a)
                elif not last:
                    fire_gather(p ^ 1, j + 2 - _S, a)

        fire_idx(0, 0)
        wait_idx(0)
        fire_gather(0, 0, 0)
        fire_gather(0, 1, 1)
        superblock(0, crow0, 0, first=True)

        def body(t, carry):
            g0 = 2 * t + 1
            crow = crow0 + g0 * _S
            superblock(1, crow, g0)
            superblock(0, crow + _S, g0 + 1)
            return carry

        lax.fori_loop(0, (n_super - 2) // 2, body, 0)

        superblock(1, crow0 + (n_super - 1) * _S, n_super - 1, last=True)
        wait_out(0)
        wait_out(1)

    return k(idx2d, packed)


def kernel(x, table):
    b = x.size
    d = table.shape[1]
    idx2d = x.reshape((b // _C, _C)).astype(jnp.int32)
    out = _gather_flat(idx2d, _pack_table(table), b, d)
    out = jax.lax.bitcast_convert_type(out, jnp.float32)
    return lax.stop_gradient(out.reshape(x.shape + (d,)))


# speedup vs baseline: 2.6303x; 2.6303x over previous
import functools

import jax
import jax.numpy as jnp
from jax import lax
from jax.experimental import pallas as pl
from jax.experimental.pallas import tpu as pltpu
from jax.experimental.pallas import tpu_sc as plsc

_NC = 2
_NS = 16
_NW = _NC * _NS

_C = 128
_S = 8
_R = 4
_G = 2


@functools.partial(jax.jit, static_argnums=(2, 3))
def _gather_flat(idx2d, table, b, d):
    b_per_w = b // _NW
    n_chunks = b_per_w // _C
    n_super = n_chunks // _S

    mesh = plsc.VectorSubcoreMesh(core_axis_name="c", subcore_axis_name="s")

    @functools.partial(
        pl.kernel,
        mesh=mesh,
        out_type=jax.ShapeDtypeStruct((b, d), jnp.float32),
        scratch_types=[
            pltpu.VMEM((2, _S, _C), jnp.int32),
            pltpu.VMEM((_R, _C, d), jnp.float32),
        ]
        + [pltpu.SemaphoreType.DMA] * (2 * _R + 2),
    )
    def k(idx_hbm, table_hbm, out_hbm, idx_v, rows, *sems):
        gs = sems[:_R]
        os_ = sems[_R:2 * _R]
        is_ = sems[2 * _R:]

        wid = lax.axis_index("s") * _NC + lax.axis_index("c")
        crow0 = wid * n_chunks

        def fire_idx(g, p):
            pltpu.async_copy(idx_hbm.at[pl.ds(crow0 + g * _S, _S)],
                             idx_v.at[p], is_[p])

        def wait_idx(p):
            pltpu.make_async_copy(idx_hbm.at[pl.ds(0, _S)], idx_v.at[p],
                                  is_[p]).wait()

        def fire_gather(p, j):
            pltpu.async_copy(table_hbm.at[idx_v.at[p, j]], rows.at[j % _R],
                             gs[j % _R])

        def wait_gather(s):
            pltpu.make_async_copy(out_hbm.at[pl.ds(0, _C)], rows.at[s],
                                  gs[s]).wait()

        def fire_out(s, crow):
            off = pl.multiple_of(crow * _C, _C)
            pltpu.async_copy(rows.at[s], out_hbm.at[pl.ds(off, _C)], os_[s])

        def wait_out(s):
            pltpu.make_async_copy(rows.at[s], out_hbm.at[pl.ds(0, _C)],
                                  os_[s]).wait()

        def superblock(p, crow, first, next_load=None):
            for j in range(_S):
                s = j % _R
                if not (first and j < _R):
                    wait_out(s)
                fire_gather(p, j)
                if not (first and j < _G):
                    ps = (j - _G) % _R
                    wait_gather(ps)
                    fire_out(ps, crow + j - _G)
                if j == _G - 1 and next_load is not None:
                    fire_idx(*next_load)

        fire_idx(0, 0)
        fire_idx(1, 1)
        wait_idx(0)
        superblock(0, crow0, first=True)
        wait_idx(1)
        superblock(1, crow0 + _S, first=False, next_load=(2, 0))

        def body(t, carry):
            g0 = 2 * t
            crow = crow0 + g0 * _S
            wait_idx(0)
            superblock(0, crow, first=False,
                       next_load=(jnp.minimum(g0 + 1, n_super - 1), 1))
            wait_idx(1)
            superblock(1, crow + _S, first=False,
                       next_load=(jnp.minimum(g0 + 2, n_super - 1), 0))
            return carry

        lax.fori_loop(1, n_super // 2, body, 0)

        for j in range(_G):
            ps = (_S - _G + j) % _R
            wait_gather(ps)
            fire_out(ps, crow0 + n_chunks - _G + j)
        wait_idx(0)
        for s in range(_R):
            wait_out(s)

    return k(idx2d, table)


def kernel(x, table):
    b = x.size
    d = table.shape[1]
    idx2d = x.reshape((b // _C, _C)).astype(jnp.int32)
    out = _gather_flat(idx2d, table, b, d)
    return lax.stop_gradient(out.reshape(x.shape + (d,)))
